# baseline (speedup 1.0000x reference)
import functools

import jax
import jax.numpy as jnp
from jax import lax
from jax.experimental import pallas as pl
from jax.experimental.pallas import tpu as pltpu
from jax.experimental.pallas import tpu_sc as plsc

_BATCH = 128
_K = 8
_ROWS_PER_WORKER = 16
_NUM_WORKERS = _BATCH // _ROWS_PER_WORKER
_IN_W = _ROWS_PER_WORKER * _K
_OUT_W = _ROWS_PER_WORKER * (_K + 1)


def _sc_body(nc, acc_hbm, draft_hbm, sub_hbm, bonus_hbm, out_hbm, part_hbm,
             acc_v, draft_v, sub_v, bonus_v, out_v, part_v):
    c = lax.axis_index("c")
    s = lax.axis_index("s")
    wid = s * nc + c

    @pl.when(wid < _NUM_WORKERS)
    def _():
        pltpu.sync_copy(acc_hbm.at[pl.ds(wid * _IN_W, _IN_W)], acc_v)
        pltpu.sync_copy(draft_hbm.at[pl.ds(wid * _IN_W, _IN_W)], draft_v)
        pltpu.sync_copy(sub_hbm.at[pl.ds(wid * _IN_W, _IN_W)], sub_v)
        pltpu.sync_copy(
            bonus_hbm.at[pl.ds(wid * _ROWS_PER_WORKER, _ROWS_PER_WORKER)],
            bonus_v)

        rows = lax.iota(jnp.int32, 16)
        in_base = rows * _K
        out_base = rows * (_K + 1)
        pref = jnp.ones((16,), jnp.int32)
        limits = jnp.zeros((16,), jnp.int32)
        acc_tot = jnp.zeros((16,), jnp.int32)
        for j in range(_K):
            idx = in_base + j
            a = plsc.load_gather(acc_v, [idx])
            d = plsc.load_gather(draft_v, [idx])
            sb = plsc.load_gather(sub_v, [idx])
            am = pref * a
            af = pref - am
            oj = am * d + af * sb + (am + af - 1)
            plsc.store_scatter(out_v, [out_base + j], oj)
            limits = limits + am
            acc_tot = acc_tot + a
            pref = am
        b = bonus_v[...]
        o_bonus = pref * b + (pref - 1)
        plsc.store_scatter(out_v, [out_base + _K], o_bonus)
        pltpu.sync_copy(out_v, out_hbm.at[pl.ds(wid * _OUT_W, _OUT_W)])

        pa = jnp.sum(acc_tot)
        pg = jnp.sum(limits)
        pvec = jnp.where(rows == 0, pa, jnp.where(rows == 1, pg, 0))
        part_v[...] = pvec
        pltpu.sync_copy(part_v, part_hbm.at[pl.ds(wid * 16, 16)])


def kernel(accepted, substitute_token_ids, draft_token_ids, bonus_token_ids,
           total_num_seqs):
    del total_num_seqs
    acc32 = accepted.astype(jnp.int32).reshape(_BATCH * _K)
    draft = draft_token_ids.reshape(_BATCH * _K)
    sub = substitute_token_ids.reshape(_BATCH * _K)
    bonus = bonus_token_ids.reshape(_BATCH)

    info = plsc.get_sparse_core_info()
    nc = info.num_cores
    mesh = plsc.VectorSubcoreMesh(core_axis_name="c", subcore_axis_name="s")

    run = pl.kernel(
        functools.partial(_sc_body, nc),
        out_type=[
            jax.ShapeDtypeStruct((_BATCH * (_K + 1),), jnp.int32),
            jax.ShapeDtypeStruct((_NUM_WORKERS * 16,), jnp.int32),
        ],
        mesh=mesh,
        scratch_types=[
            pltpu.VMEM((_IN_W,), jnp.int32),
            pltpu.VMEM((_IN_W,), jnp.int32),
            pltpu.VMEM((_IN_W,), jnp.int32),
            pltpu.VMEM((_ROWS_PER_WORKER,), jnp.int32),
            pltpu.VMEM((_OUT_W,), jnp.int32),
            pltpu.VMEM((16,), jnp.int32),
        ],
        compiler_params=pltpu.CompilerParams(needs_layout_passes=False),
        name="spec_decode_sampler_patch_sc",
    )
    out_flat, partials = run(acc32, draft, sub, bonus)

    out_tokens = out_flat.reshape(_BATCH, _K + 1)
    partials = partials.reshape(_NUM_WORKERS, 16)
    num_accepted = partials[:, 0].sum()
    num_good_draft = partials[:, 1].sum()
    num_emitted = num_good_draft + jnp.int32(_BATCH)
    return out_tokens, num_accepted, num_good_draft, num_emitted


# speedup vs baseline: 1.0377x; 1.0377x over previous
import functools

import jax
import jax.numpy as jnp
from jax import lax
from jax.experimental import pallas as pl
from jax.experimental.pallas import tpu as pltpu
from jax.experimental.pallas import tpu_sc as plsc

_BATCH = 128
_K = 8
_RPW = 16
_NW = _BATCH // _RPW
_SEG = _RPW * _K
_PACK = 3 * _SEG + _RPW
_OUT_W = _RPW * (_K + 1)


def _sc_body(nc, packed_hbm, out_hbm, part_hbm, buf, out_v, part_v):
    c = lax.axis_index("c")
    s = lax.axis_index("s")
    wid = s * nc + c

    @pl.when(wid < _NW)
    def _():
        pltpu.sync_copy(packed_hbm.at[pl.ds(wid * _PACK, _PACK)], buf)

        rows = lax.iota(jnp.int32, 16)
        in_base = rows * _K
        out_base = rows * (_K + 1)

        def step(j, carry):
            pref, limits, acc_tot = carry
            idx = in_base + j
            a = plsc.load_gather(buf, [idx])
            d = plsc.load_gather(buf, [idx + _SEG])
            sb = plsc.load_gather(buf, [idx + 2 * _SEG])
            am = pref * a
            af = pref - am
            oj = am * d + af * sb + (am + af - 1)
            plsc.store_scatter(out_v, [out_base + j], oj)
            return am, limits + am, acc_tot + a

        ones = jnp.ones((16,), jnp.int32)
        zeros = jnp.zeros((16,), jnp.int32)
        pref, limits, acc_tot = lax.fori_loop(0, _K, step, (ones, zeros, zeros))

        b = buf[pl.ds(3 * _SEG, _RPW)]
        o_bonus = pref * b + (pref - 1)
        plsc.store_scatter(out_v, [out_base + _K], o_bonus)
        pltpu.sync_copy(out_v, out_hbm.at[pl.ds(wid * _OUT_W, _OUT_W)])

        pa = jnp.sum(acc_tot)
        pg = jnp.sum(limits)
        pvec = jnp.where(rows == 0, pa, jnp.where(rows == 1, pg, 0))
        part_v[...] = pvec
        pltpu.sync_copy(part_v, part_hbm.at[pl.ds(wid * 16, 16)])


def kernel(accepted, substitute_token_ids, draft_token_ids, bonus_token_ids,
           total_num_seqs):
    del total_num_seqs
    packed = jnp.concatenate(
        [
            accepted.astype(jnp.int32).reshape(_NW, _SEG),
            draft_token_ids.reshape(_NW, _SEG),
            substitute_token_ids.reshape(_NW, _SEG),
            bonus_token_ids.reshape(_NW, _RPW),
        ],
        axis=1,
    ).reshape(_NW * _PACK)

    info = plsc.get_sparse_core_info()
    nc = info.num_cores
    mesh = plsc.VectorSubcoreMesh(core_axis_name="c", subcore_axis_name="s")

    run = pl.kernel(
        functools.partial(_sc_body, nc),
        out_type=[
            jax.ShapeDtypeStruct((_BATCH * (_K + 1),), jnp.int32),
            jax.ShapeDtypeStruct((_NW * 16,), jnp.int32),
        ],
        mesh=mesh,
        scratch_types=[
            pltpu.VMEM((_PACK,), jnp.int32),
            pltpu.VMEM((_OUT_W,), jnp.int32),
            pltpu.VMEM((16,), jnp.int32),
        ],
        compiler_params=pltpu.CompilerParams(needs_layout_passes=False),
        name="spec_decode_sampler_patch_sc",
    )
    out_flat, partials = run(packed)

    out_tokens = out_flat.reshape(_BATCH, _K + 1)
    partials = partials.reshape(_NW, 16)
    num_accepted = partials[:, 0].sum()
    num_good_draft = partials[:, 1].sum()
    num_emitted = num_good_draft + jnp.int32(_BATCH)
    return out_tokens, num_accepted, num_good_draft, num_emitted


# speedup vs baseline: 3.3941x; 3.2707x over previous
import jax
import jax.numpy as jnp
from jax import lax
from jax.experimental import pallas as pl
from jax.experimental.pallas import tpu as pltpu

_B, _K = 128, 8
_N = _B * _K


def _tc_body(in_ref, out_ref, na_ref, ng_ref, ne_ref):
    x = in_ref[...].reshape(32, 128)
    a = x[0:8, :]
    sub = x[8:16, :]
    draft = x[16:24, :]
    bonus = x[24:32, :]
    pos = lax.broadcasted_iota(jnp.int32, (8, 128), 1) & (_K - 1)

    def shiftg(x, d):
        return jnp.where(pos >= d, pltpu.roll(x, d, 1), 0)

    bad = 1 - a
    s = bad | shiftg(bad, 1)
    s = s | shiftg(s, 2)
    incl = s | shiftg(s, 4)
    acc_mask = 1 - incl
    after = bad * (1 - shiftg(incl, 1))

    out8 = acc_mask * draft + after * sub + (acc_mask + after - 1)
    bflag = jnp.where(pos == _K - 1, acc_mask, 0)
    obp = bflag * bonus + bflag - 1
    out_ref[...] = jnp.concatenate([out8, obp], axis=0).reshape(2 * _N)

    na_ref[0, 0] = jnp.sum(a)
    ng = jnp.sum(acc_mask)
    ng_ref[0, 0] = ng
    ne_ref[0, 0] = ng + _B


def kernel(accepted, substitute_token_ids, draft_token_ids, bonus_token_ids,
           total_num_seqs):
    del total_num_seqs
    packed = jnp.concatenate([
        accepted.astype(jnp.int32).reshape(_N),
        substitute_token_ids.reshape(_N),
        draft_token_ids.reshape(_N),
        jnp.broadcast_to(bonus_token_ids, (_B, _K)).reshape(_N),
    ])
    out, na, ng, ne = pl.pallas_call(
        _tc_body,
        out_shape=[
            jax.ShapeDtypeStruct((2 * _N,), jnp.int32),
            jax.ShapeDtypeStruct((1, 1), jnp.int32),
            jax.ShapeDtypeStruct((1, 1), jnp.int32),
            jax.ShapeDtypeStruct((1, 1), jnp.int32),
        ],
        out_specs=[
            pl.BlockSpec(memory_space=pltpu.VMEM),
            pl.BlockSpec(memory_space=pltpu.SMEM),
            pl.BlockSpec(memory_space=pltpu.SMEM),
            pl.BlockSpec(memory_space=pltpu.SMEM),
        ],
        name="spec_decode_sampler_patch_tc2",
    )(packed)
    out_tokens = jnp.concatenate(
        [out[:_N].reshape(_B, _K), out[_N:].reshape(_B, _K)[:, _K - 1:]],
        axis=1)
    return out_tokens, na[0, 0], ng[0, 0], ne[0, 0]


# speedup vs baseline: 7.9899x; 2.3540x over previous
import jax
import jax.numpy as jnp
from jax import lax
from jax.experimental import pallas as pl
from jax.experimental.pallas import tpu as pltpu

_B, _K = 128, 8


def _tc_body(acc_ref, sub_ref, draft_ref, bonus_ref, out_ref, na_ref, ng_ref,
             ne_ref):
    a = acc_ref[...].astype(jnp.int32)
    sub = sub_ref[...]
    draft = draft_ref[...]

    pref = jnp.ones((1, _B), jnp.int32)
    limits = jnp.zeros((1, _B), jnp.int32)
    outs = []
    for j in range(_K):
        aj = a[j:j + 1, :]
        am = pref * aj
        af = pref - am
        outs.append(am * draft[j:j + 1, :] + af * sub[j:j + 1, :]
                    + (am + af - 1))
        limits = limits + am
        pref = am
    bonus = bonus_ref[...]
    outs.append(pref * bonus + (pref - 1))
    out_ref[...] = jnp.concatenate(outs, axis=0)

    na_ref[0, 0] = jnp.sum(a)
    ng = jnp.sum(limits)
    ng_ref[0, 0] = ng
    ne_ref[0, 0] = ng + _B


def kernel(accepted, substitute_token_ids, draft_token_ids, bonus_token_ids,
           total_num_seqs):
    del total_num_seqs
    out_t, na, ng, ne = pl.pallas_call(
        _tc_body,
        out_shape=[
            jax.ShapeDtypeStruct((_K + 1, _B), jnp.int32),
            jax.ShapeDtypeStruct((1, 1), jnp.int32),
            jax.ShapeDtypeStruct((1, 1), jnp.int32),
            jax.ShapeDtypeStruct((1, 1), jnp.int32),
        ],
        out_specs=[
            pl.BlockSpec(memory_space=pltpu.VMEM),
            pl.BlockSpec(memory_space=pltpu.SMEM),
            pl.BlockSpec(memory_space=pltpu.SMEM),
            pl.BlockSpec(memory_space=pltpu.SMEM),
        ],
        name="spec_decode_sampler_patch_tc3",
    )(accepted.T, substitute_token_ids.T, draft_token_ids.T,
      bonus_token_ids.T)
    return out_t.T, na[0, 0], ng[0, 0], ne[0, 0]


# speedup vs baseline: 8.1200x; 1.0163x over previous
import jax
import jax.numpy as jnp
from jax import lax
from jax.experimental import pallas as pl
from jax.experimental.pallas import tpu as pltpu

_B, _K = 128, 8


def _tc_body(acc_ref, sub_ref, draft_ref, bonus_ref, out_ref, na_ref, ng_ref,
             ne_ref):
    a = acc_ref[...].astype(jnp.int32)
    sub = sub_ref[...]
    draft = draft_ref[...]

    pref = jnp.ones((1, _B), jnp.int32)
    limits = jnp.zeros((1, _B), jnp.int32)
    outs = []
    for j in range(_K):
        aj = a[j:j + 1, :]
        am = pref * aj
        af = pref - am
        outs.append(am * draft[j:j + 1, :] + af * sub[j:j + 1, :]
                    + (am + af - 1))
        limits = limits + am
        pref = am
    bonus = bonus_ref[...]
    outs.append(pref * bonus + (pref - 1))
    out_ref[...] = jnp.concatenate(outs, axis=0)

    na_ref[0, 0] = jnp.sum(a)
    ng = jnp.sum(limits)
    ng_ref[0, 0] = ng
    ne_ref[0, 0] = ng + _B


def kernel(accepted, substitute_token_ids, draft_token_ids, bonus_token_ids,
           total_num_seqs):
    del total_num_seqs
    out_t, na, ng, ne = pl.pallas_call(
        _tc_body,
        out_shape=[
            jax.ShapeDtypeStruct((_K + 1, _B), jnp.int32),
            jax.ShapeDtypeStruct((1, 1), jnp.int32),
            jax.ShapeDtypeStruct((1, 1), jnp.int32),
            jax.ShapeDtypeStruct((1, 1), jnp.int32),
        ],
        out_specs=[
            pl.BlockSpec(memory_space=pltpu.VMEM),
            pl.BlockSpec(memory_space=pltpu.SMEM),
            pl.BlockSpec(memory_space=pltpu.SMEM),
            pl.BlockSpec(memory_space=pltpu.SMEM),
        ],
        name="spec_decode_sampler_patch_tc3",
    )(accepted.view(jnp.int8).T,
      substitute_token_ids.T, draft_token_ids.T, bonus_token_ids.T)
    return out_t.T, na[0, 0], ng[0, 0], ne[0, 0]
